# baseline (speedup 1.0000x reference)
import functools

import jax
import jax.numpy as jnp
from jax import lax
from jax.experimental import pallas as pl
from jax.experimental.pallas import tpu as pltpu
from jax.experimental.pallas import tpu_sc as plsc

N = 10000
E = 320000
NFEAT = 128
NHID = 64

NC = 2
NS = 16
NW = NC * NS
E_PER_W = E // NW
E_PER_T = E // NS
NHALF = NHID // 2
EDGE_BLK = 1000
N_BLKS = E_PER_T // EDGE_BLK
ROW_TILES = 10
ROWS_PER_TILE = N // ROW_TILES

_UNTILED = pltpu.CompilerParams(use_tc_tiling_on_sc=False)


@functools.cache
def _mesh():
    return plsc.VectorSubcoreMesh(core_axis_name="c", subcore_axis_name="s",
                                  num_cores=NC, num_subcores=NS)


def _sc_degree_body(eif_hbm, zeros1_hbm, deg_out, idx_v, ones_v, stage_v,
                    shared_deg):
    cid = lax.axis_index("c")
    sid = lax.axis_index("s")
    wid = cid * NS + sid

    @pl.when(sid < ROW_TILES)
    def _():
        sl = pl.ds(sid * ROWS_PER_TILE, ROWS_PER_TILE)
        pltpu.sync_copy(zeros1_hbm.at[sl], stage_v)
        pltpu.sync_copy(stage_v, shared_deg.at[sl])

    def fill(i, carry):
        ones_v[pl.ds(i * 16, 16)] = jnp.ones((16,), jnp.float32)
        return carry
    lax.fori_loop(0, E_PER_W // 16, fill, 0)

    pltpu.sync_copy(eif_hbm.at[pl.ds(2 * E + wid * E_PER_W, E_PER_W)], idx_v)

    plsc.subcore_barrier()
    pltpu.sync_copy(ones_v, shared_deg.at[idx_v], add=True)
    plsc.subcore_barrier()

    @pl.when(sid < ROW_TILES)
    def _():
        sl = pl.ds(sid * ROWS_PER_TILE, ROWS_PER_TILE)
        osl = pl.ds(cid * N + sid * ROWS_PER_TILE, ROWS_PER_TILE)
        pltpu.sync_copy(shared_deg.at[sl], stage_v)
        pltpu.sync_copy(stage_v, deg_out.at[osl])


@functools.cache
def _sc_degree():
    return pl.kernel(
        _sc_degree_body,
        out_type=jax.ShapeDtypeStruct((NC * N,), jnp.float32),
        mesh=_mesh(),
        scratch_types=[
            pltpu.VMEM((E_PER_W,), jnp.int32),
            pltpu.VMEM((E_PER_W,), jnp.float32),
            pltpu.VMEM((ROWS_PER_TILE,), jnp.float32),
            pltpu.VMEM_SHARED((N,), jnp.float32),
        ],
        compiler_params=_UNTILED,
    )


def _sc_aggregate_body(g_hbm, eif_hbm, zeros2_hbm, agg_out,
                       src_all, dst_v0, dst_v1, rows_v0, rows_v1,
                       shared_a, shared_b, gsem0, gsem1, ssem0):
    cid = lax.axis_index("c")
    sid = lax.axis_index("s")

    dst_v = (dst_v0, dst_v1)
    rows_v = (rows_v0, rows_v1)
    gsem = (gsem0, gsem1)

    @pl.when(sid < ROW_TILES)
    def _():
        pltpu.sync_copy(zeros2_hbm.at[pl.ds(0, ROWS_PER_TILE)], rows_v0)

    @pl.when(sid < ROW_TILES)
    def _():
        sl = pl.ds(sid * ROWS_PER_TILE, ROWS_PER_TILE)
        pltpu.sync_copy(rows_v0, shared_a.at[sl])

    @pl.when(sid >= NS - ROW_TILES)
    def _():
        sl = pl.ds((sid - (NS - ROW_TILES)) * ROWS_PER_TILE, ROWS_PER_TILE)
        pltpu.sync_copy(rows_v0, shared_b.at[sl])

    plsc.subcore_barrier()

    base_e = sid * E_PER_T

    pltpu.sync_copy(eif_hbm.at[pl.ds(cid * E + base_e, E_PER_T)], src_all)

    def isl(i):
        return pl.ds(i * EDGE_BLK, EDGE_BLK)

    def dsl(i):
        return pl.ds(2 * E + base_e + i * EDGE_BLK, EDGE_BLK)

    def pipeline(acc):
        pltpu.sync_copy(eif_hbm.at[dsl(0)], dst_v[0])
        gd = [pltpu.async_copy(g_hbm.at[src_all.at[isl(0)]], rows_v[0],
                               gsem[0]), None]
        sd = None
        for i in range(N_BLKS):
            b = i % 2
            gd[b].wait()
            if sd is not None:
                sd.wait()
            sd = pltpu.async_copy(rows_v[b], acc.at[dst_v[b]],
                                  ssem0, add=True)
            if i + 1 < N_BLKS:
                nb = (i + 1) % 2
                pltpu.sync_copy(eif_hbm.at[dsl(i + 1)], dst_v[nb])
                gd[nb] = pltpu.async_copy(g_hbm.at[src_all.at[isl(i + 1)]],
                                          rows_v[nb], gsem[nb])
        sd.wait()

    @pl.when(sid % 2 == 0)
    def _():
        pipeline(shared_a)

    @pl.when(sid % 2 == 1)
    def _():
        pipeline(shared_b)

    plsc.subcore_barrier()

    @pl.when(sid < ROW_TILES)
    def _():
        sl = pl.ds(sid * ROWS_PER_TILE, ROWS_PER_TILE)
        csl = pl.ds(cid * NHALF, NHALF)
        pltpu.sync_copy(shared_a.at[sl], rows_v0)
        pltpu.sync_copy(rows_v0, agg_out.at[0, sl, csl])
        pltpu.sync_copy(shared_b.at[sl], rows_v1)
        pltpu.sync_copy(rows_v1, agg_out.at[1, sl, csl])


@functools.cache
def _sc_aggregate():
    return pl.kernel(
        _sc_aggregate_body,
        out_type=jax.ShapeDtypeStruct((2, N, 128), jnp.float32),
        mesh=_mesh(),
        scratch_types=[
            pltpu.VMEM((E_PER_T,), jnp.int32),
            pltpu.VMEM((EDGE_BLK,), jnp.int32),
            pltpu.VMEM((EDGE_BLK,), jnp.int32),
            pltpu.VMEM((EDGE_BLK, NHALF), jnp.float32),
            pltpu.VMEM((EDGE_BLK, NHALF), jnp.float32),
            pltpu.VMEM_SHARED((N, NHALF), jnp.float32),
            pltpu.VMEM_SHARED((N, NHALF), jnp.float32),
            pltpu.SemaphoreType.DMA,
            pltpu.SemaphoreType.DMA,
            pltpu.SemaphoreType.DMA,
        ],
        compiler_params=_UNTILED,
    )


def _tc_edgeprep_body(ei_ref, out_ref):
    s4 = ei_ref[0] * 4
    out_ref[pl.ds(0, E)] = s4
    out_ref[pl.ds(E, E)] = s4 + 1
    out_ref[pl.ds(2 * E, E)] = ei_ref[1]


def _tc_edgeprep(ei):
    return pl.pallas_call(
        _tc_edgeprep_body,
        out_shape=jax.ShapeDtypeStruct((3 * E,), jnp.int32),
    )(ei)


def _tc_matmul_body(x_ref, w1_ref, h_ref):
    h = jnp.dot(x_ref[...], w1_ref[...], preferred_element_type=jnp.float32)
    h_ref[...] = jnp.concatenate(
        [h, jnp.zeros((N, 128 - NHID), jnp.float32)], axis=1)


def _tc_matmul(x, w1):
    return pl.pallas_call(
        _tc_matmul_body,
        out_shape=jax.ShapeDtypeStruct((N, 128), jnp.float32),
    )(x, w1)


def _dinv_cols(deg_ref, ncols):
    d = deg_ref[pl.ds(0, N)] + deg_ref[pl.ds(N, N)]
    dinv_row = lax.rsqrt(jnp.maximum(d, 1.0))[None, :]
    ones_row = jnp.ones((1, ncols), jnp.float32)
    return lax.dot_general(dinv_row, ones_row, (((0,), (0,)), ((), ())),
                           preferred_element_type=jnp.float32)


def _tc_prescale_body(deg_ref, h_ref, g_ref):
    g = h_ref[...][:, :NHID] * _dinv_cols(deg_ref, NHID)
    g_ref[...] = jnp.concatenate(
        [g, jnp.zeros((N, 128 - NHID), jnp.float32)], axis=1)


def _tc_prescale(deg_flat, h):
    return pl.pallas_call(
        _tc_prescale_body,
        out_shape=jax.ShapeDtypeStruct((N, 128), jnp.float32),
    )(deg_flat, h)


def _tc_heads_body(agg_ref, deg_ref, b1_ref, wc_ref, ws_ref, bc_ref, bs_ref,
                   z_ref, y_ref, s_ref):
    agg = (agg_ref[0] + agg_ref[1])[:, :NHID]
    z = agg * _dinv_cols(deg_ref, NHID) + b1_ref[...]
    z_ref[...] = z
    y_ref[...] = jnp.sum(z * wc_ref[...], axis=1) + bc_ref[0, 0]
    s_ref[...] = jnp.sum(z * ws_ref[...], axis=1) + bs_ref[0, 0]


def _tc_heads(agg, deg_flat, b1, wc, ws, bc, bs):
    return pl.pallas_call(
        _tc_heads_body,
        out_shape=(
            jax.ShapeDtypeStruct((N, NHID), jnp.float32),
            jax.ShapeDtypeStruct((N,), jnp.float32),
            jax.ShapeDtypeStruct((N,), jnp.float32),
        ),
    )(agg, deg_flat, b1, wc, ws, bc, bs)


def kernel(x, edge_index, W1, b1, Wc, bc, Ws, bs):
    if edge_index.dtype != jnp.int32:
        edge_index = edge_index.astype(jnp.int32)

    if edge_index.dtype != jnp.int32:
        edge_index = edge_index.astype(jnp.int32)
    ei3 = _tc_edgeprep(edge_index)
    zeros1 = jnp.zeros((N,), jnp.float32)
    zeros2 = jnp.zeros((N, NHALF), jnp.float32)

    h = _tc_matmul(x, W1)
    deg_flat = _sc_degree()(ei3, zeros1)
    g = _tc_prescale(deg_flat, h)
    g4 = g.reshape(4 * N, 32)
    agg_parts = _sc_aggregate()(g4, ei3, zeros2)

    z, y, s = _tc_heads(agg_parts, deg_flat, b1[None, :],
                        Wc.reshape(1, NHID), Ws.reshape(1, NHID),
                        bc[None, :], bs[None, :])
    return (z, y[:, None], s[:, None])


# speedup vs baseline: 1.0992x; 1.0992x over previous
import functools

import jax
import jax.numpy as jnp
from jax import lax
from jax.experimental import pallas as pl
from jax.experimental.pallas import tpu as pltpu
from jax.experimental.pallas import tpu_sc as plsc

N = 10000
E = 320000
NFEAT = 128
NHID = 64

NC = 2
NS = 16
NW = NC * NS
E_PER_W = E // NW
E_PER_T = E // NS
NHALF = NHID // 2
EDGE_BLK = 1000
N_BLKS = E_PER_T // EDGE_BLK
ROW_TILES = 10
ROWS_PER_TILE = N // ROW_TILES

_UNTILED = pltpu.CompilerParams(use_tc_tiling_on_sc=False)


@functools.cache
def _mesh():
    return plsc.VectorSubcoreMesh(core_axis_name="c", subcore_axis_name="s",
                                  num_cores=NC, num_subcores=NS)


def _sc_degree_body(eif_hbm, zeros1_hbm, deg_out, idx_v, ones_v, stage_v,
                    shared_deg):
    cid = lax.axis_index("c")
    sid = lax.axis_index("s")
    wid = cid * NS + sid

    @pl.when(sid < ROW_TILES)
    def _():
        sl = pl.ds(sid * ROWS_PER_TILE, ROWS_PER_TILE)
        pltpu.sync_copy(zeros1_hbm.at[sl], stage_v)
        pltpu.sync_copy(stage_v, shared_deg.at[sl])

    def fill(i, carry):
        ones_v[pl.ds(i * 16, 16)] = jnp.ones((16,), jnp.float32)
        return carry
    lax.fori_loop(0, E_PER_W // 16, fill, 0)

    pltpu.sync_copy(eif_hbm.at[pl.ds(2 * E + wid * E_PER_W, E_PER_W)], idx_v)

    plsc.subcore_barrier()
    pltpu.sync_copy(ones_v, shared_deg.at[idx_v], add=True)
    plsc.subcore_barrier()

    @pl.when(sid < ROW_TILES)
    def _():
        sl = pl.ds(sid * ROWS_PER_TILE, ROWS_PER_TILE)
        osl = pl.ds(cid * N + sid * ROWS_PER_TILE, ROWS_PER_TILE)
        pltpu.sync_copy(shared_deg.at[sl], stage_v)
        pltpu.sync_copy(stage_v, deg_out.at[osl])


@functools.cache
def _sc_degree():
    return pl.kernel(
        _sc_degree_body,
        out_type=jax.ShapeDtypeStruct((NC * N,), jnp.float32),
        mesh=_mesh(),
        scratch_types=[
            pltpu.VMEM((E_PER_W,), jnp.int32),
            pltpu.VMEM((E_PER_W,), jnp.float32),
            pltpu.VMEM((ROWS_PER_TILE,), jnp.float32),
            pltpu.VMEM_SHARED((N,), jnp.float32),
        ],
        compiler_params=_UNTILED,
    )


def _sc_aggregate_body(g_hbm, eif_hbm, zeros2_hbm, agg_out,
                       src_all, dst_all, rows_v0, rows_v1,
                       shared_agg, gsem0, gsem1, ssem0):
    cid = lax.axis_index("c")
    sid = lax.axis_index("s")

    rows_v = (rows_v0, rows_v1)
    gsem = (gsem0, gsem1)

    @pl.when(sid < ROW_TILES)
    def _():
        sl = pl.ds(sid * ROWS_PER_TILE, ROWS_PER_TILE)
        pltpu.sync_copy(zeros2_hbm.at[sl], rows_v0)
        pltpu.sync_copy(rows_v0, shared_agg.at[sl])

    plsc.subcore_barrier()

    base_e = sid * E_PER_T

    pltpu.sync_copy(eif_hbm.at[pl.ds(cid * E + base_e, E_PER_T)], src_all)
    pltpu.sync_copy(eif_hbm.at[pl.ds(2 * E + base_e, E_PER_T)], dst_all)

    def isl(i):
        return pl.ds(i * EDGE_BLK, EDGE_BLK)

    gd = [pltpu.async_copy(g_hbm.at[src_all.at[isl(0)]], rows_v[0],
                           gsem[0]), None]
    sd = None

    for i in range(N_BLKS):
        b = i % 2
        gd[b].wait()
        if sd is not None:
            sd.wait()
        sd = pltpu.async_copy(rows_v[b], shared_agg.at[dst_all.at[isl(i)]],
                              ssem0, add=True)
        if i + 1 < N_BLKS:
            nb = (i + 1) % 2
            gd[nb] = pltpu.async_copy(g_hbm.at[src_all.at[isl(i + 1)]],
                                      rows_v[nb], gsem[nb])
    sd.wait()

    plsc.subcore_barrier()

    @pl.when(sid < ROW_TILES)
    def _():
        sl = pl.ds(sid * ROWS_PER_TILE, ROWS_PER_TILE)
        pltpu.sync_copy(shared_agg.at[sl], rows_v0)
        pltpu.sync_copy(rows_v0,
                        agg_out.at[sl, pl.ds(cid * NHALF, NHALF)])


@functools.cache
def _sc_aggregate():
    return pl.kernel(
        _sc_aggregate_body,
        out_type=jax.ShapeDtypeStruct((N, 128), jnp.float32),
        mesh=_mesh(),
        scratch_types=[
            pltpu.VMEM((E_PER_T,), jnp.int32),
            pltpu.VMEM((E_PER_T,), jnp.int32),
            pltpu.VMEM((EDGE_BLK, NHALF), jnp.float32),
            pltpu.VMEM((EDGE_BLK, NHALF), jnp.float32),
            pltpu.VMEM_SHARED((N, NHALF), jnp.float32),
            pltpu.SemaphoreType.DMA,
            pltpu.SemaphoreType.DMA,
            pltpu.SemaphoreType.DMA,
        ],
        compiler_params=_UNTILED,
    )


def _tc_edgeprep_body(ei_ref, out_ref):
    s4 = ei_ref[0] * 4
    out_ref[pl.ds(0, E)] = s4
    out_ref[pl.ds(E, E)] = s4 + 1
    out_ref[pl.ds(2 * E, E)] = ei_ref[1]


def _tc_edgeprep(ei):
    return pl.pallas_call(
        _tc_edgeprep_body,
        out_shape=jax.ShapeDtypeStruct((3 * E,), jnp.int32),
    )(ei)


def _tc_matmul_body(x_ref, w1_ref, h_ref):
    h = jnp.dot(x_ref[...], w1_ref[...], preferred_element_type=jnp.float32)
    h_ref[...] = jnp.concatenate(
        [h, jnp.zeros((N, 128 - NHID), jnp.float32)], axis=1)


def _tc_matmul(x, w1):
    return pl.pallas_call(
        _tc_matmul_body,
        out_shape=jax.ShapeDtypeStruct((N, 128), jnp.float32),
    )(x, w1)


def _dinv_cols(deg_ref, ncols):
    d = deg_ref[pl.ds(0, N)] + deg_ref[pl.ds(N, N)]
    dinv_row = lax.rsqrt(jnp.maximum(d, 1.0))[None, :]
    ones_row = jnp.ones((1, ncols), jnp.float32)
    return lax.dot_general(dinv_row, ones_row, (((0,), (0,)), ((), ())),
                           preferred_element_type=jnp.float32)


def _tc_prescale_body(deg_ref, h_ref, g_ref):
    g = h_ref[...][:, :NHID] * _dinv_cols(deg_ref, NHID)
    g_ref[...] = jnp.concatenate(
        [g, jnp.zeros((N, 128 - NHID), jnp.float32)], axis=1)


def _tc_prescale(deg_flat, h):
    return pl.pallas_call(
        _tc_prescale_body,
        out_shape=jax.ShapeDtypeStruct((N, 128), jnp.float32),
    )(deg_flat, h)


def _tc_heads_body(agg_ref, deg_ref, b1_ref, wc_ref, ws_ref, bc_ref, bs_ref,
                   z_ref, y_ref, s_ref):
    agg = agg_ref[...][:, :NHID]
    z = agg * _dinv_cols(deg_ref, NHID) + b1_ref[...]
    z_ref[...] = z
    y_ref[...] = jnp.sum(z * wc_ref[...], axis=1) + bc_ref[0, 0]
    s_ref[...] = jnp.sum(z * ws_ref[...], axis=1) + bs_ref[0, 0]


def _tc_heads(agg, deg_flat, b1, wc, ws, bc, bs):
    return pl.pallas_call(
        _tc_heads_body,
        out_shape=(
            jax.ShapeDtypeStruct((N, NHID), jnp.float32),
            jax.ShapeDtypeStruct((N,), jnp.float32),
            jax.ShapeDtypeStruct((N,), jnp.float32),
        ),
    )(agg, deg_flat, b1, wc, ws, bc, bs)


def kernel(x, edge_index, W1, b1, Wc, bc, Ws, bs):
    if edge_index.dtype != jnp.int32:
        edge_index = edge_index.astype(jnp.int32)

    if edge_index.dtype != jnp.int32:
        edge_index = edge_index.astype(jnp.int32)
    ei3 = _tc_edgeprep(edge_index)
    zeros1 = jnp.zeros((N,), jnp.float32)
    zeros2 = jnp.zeros((N, NHALF), jnp.float32)

    h = _tc_matmul(x, W1)
    deg_flat = _sc_degree()(ei3, zeros1)
    g = _tc_prescale(deg_flat, h)
    g4 = g.reshape(4 * N, 32)
    agg_parts = _sc_aggregate()(g4, ei3, zeros2)

    z, y, s = _tc_heads(agg_parts, deg_flat, b1[None, :],
                        Wc.reshape(1, NHID), Ws.reshape(1, NHID),
                        bc[None, :], bs[None, :])
    return (z, y[:, None], s[:, None])
